# baseline (speedup 1.0000x reference)
import functools

import jax
import jax.numpy as jnp
from jax.experimental import pallas as pl
from jax.experimental.pallas import tpu as pltpu

_T_R = 512
_T_B = 256


def _router_kernel(x_ref, wr_ref, probs_ref, idx_ref, w_ref, *, num_experts):
    xb = x_ref[...]
    logits = jnp.dot(xb, wr_ref[...], preferred_element_type=jnp.float32)
    m = jnp.max(logits, axis=-1, keepdims=True)
    p = jnp.exp(logits - m)
    probs = p / jnp.sum(p, axis=-1, keepdims=True)
    probs_ref[...] = probs
    iota = jax.lax.broadcasted_iota(jnp.int32, probs.shape, 1)
    m1 = jnp.max(probs, axis=-1, keepdims=True)
    i1 = jnp.min(jnp.where(probs == m1, iota, num_experts), axis=-1,
                 keepdims=True)
    masked = jnp.where(iota == i1, -1.0, probs)
    m2 = jnp.max(masked, axis=-1, keepdims=True)
    i2 = jnp.min(jnp.where(masked == m2, iota, num_experts), axis=-1,
                 keepdims=True)
    s = m1 + m2
    idx_ref[...] = jnp.concatenate([i1, i2], axis=-1)
    w_ref[...] = jnp.concatenate([m1 / s, m2 / s], axis=-1)


def _ffn_kernel(be_ref, x_ref, w1_ref, b1_ref, w2_ref, b2_ref, wg_ref, o_ref):
    g = pl.program_id(0)

    @pl.when(be_ref[g] >= 0)
    def _():
        xb = x_ref[...]
        h = jnp.dot(xb, w1_ref[0], preferred_element_type=jnp.float32)
        h = h + b1_ref[0]
        a = 0.5 * h * (1.0 + jax.lax.erf(h * 0.7071067811865476))
        y = jnp.dot(a.astype(jnp.bfloat16), w2_ref[0],
                    preferred_element_type=jnp.float32)
        y = y + b2_ref[0]
        o_ref[...] = y * wg_ref[...]


def kernel(x, Wr, W1, b1, W2, b2):
    B, S, D = x.shape
    E = Wr.shape[1]
    H = W1.shape[2]
    K = 2
    N = B * S
    P = N * K
    P_MAX = P + E * _T_B
    G_MAX = P_MAX // _T_B

    x2d = x.reshape(N, D)

    probs, sel, w = pl.pallas_call(
        functools.partial(_router_kernel, num_experts=E),
        grid=(N // _T_R,),
        in_specs=[
            pl.BlockSpec((_T_R, D), lambda i: (i, 0)),
            pl.BlockSpec((D, E), lambda i: (0, 0)),
        ],
        out_specs=[
            pl.BlockSpec((_T_R, E), lambda i: (i, 0)),
            pl.BlockSpec((_T_R, K), lambda i: (i, 0)),
            pl.BlockSpec((_T_R, K), lambda i: (i, 0)),
        ],
        out_shape=[
            jax.ShapeDtypeStruct((N, E), jnp.float32),
            jax.ShapeDtypeStruct((N, K), jnp.int32),
            jax.ShapeDtypeStruct((N, K), jnp.float32),
        ],
    )(x2d, Wr)

    e_flat = sel.reshape(P)
    onehot = (e_flat[:, None] == jnp.arange(E, dtype=jnp.int32)[None, :]
              ).astype(jnp.int32)
    ranks_all = jnp.cumsum(onehot, axis=0) - onehot
    rank = jnp.take_along_axis(ranks_all, e_flat[:, None], axis=1)[:, 0]
    counts = jnp.sum(onehot, axis=0)
    padded = ((counts + _T_B - 1) // _T_B) * _T_B
    ends = jnp.cumsum(padded)
    offs = ends - padded
    pos = offs[e_flat] + rank

    blk_start = jnp.arange(G_MAX, dtype=jnp.int32) * _T_B
    be = jnp.sum((blk_start[:, None] >= ends[None, :]).astype(jnp.int32),
                 axis=1)
    block_expert = jnp.where(be < E, be, -1).astype(jnp.int32)

    row_token = jnp.zeros((P_MAX,), jnp.int32).at[pos].set(
        jnp.arange(P, dtype=jnp.int32) // K)
    wg = jnp.zeros((P_MAX,), jnp.float32).at[pos].set(w.reshape(P))

    xg = x2d.astype(jnp.bfloat16)[row_token]

    grid_spec = pltpu.PrefetchScalarGridSpec(
        num_scalar_prefetch=1,
        grid=(G_MAX,),
        in_specs=[
            pl.BlockSpec((_T_B, D), lambda g, be: (g, 0)),
            pl.BlockSpec((1, D, H),
                         lambda g, be: (jnp.maximum(be[g], 0), 0, 0)),
            pl.BlockSpec((1, 1, H),
                         lambda g, be: (jnp.maximum(be[g], 0), 0, 0)),
            pl.BlockSpec((1, H, D),
                         lambda g, be: (jnp.maximum(be[g], 0), 0, 0)),
            pl.BlockSpec((1, 1, D),
                         lambda g, be: (jnp.maximum(be[g], 0), 0, 0)),
            pl.BlockSpec((_T_B, 1), lambda g, be: (g, 0)),
        ],
        out_specs=pl.BlockSpec((_T_B, D), lambda g, be: (g, 0)),
    )
    yg = pl.pallas_call(
        _ffn_kernel,
        grid_spec=grid_spec,
        out_shape=jax.ShapeDtypeStruct((P_MAX, D), jnp.float32),
    )(block_expert, xg, W1.astype(jnp.bfloat16), b1.reshape(E, 1, H),
      W2.astype(jnp.bfloat16), b2.reshape(E, 1, D), wg[:, None])

    pos2 = pos.reshape(N, K)
    out2d = yg[pos2[:, 0]] + yg[pos2[:, 1]]

    return (out2d.reshape(B, S, D), probs.reshape(B, S, E),
            sel.reshape(B, S, K), w.reshape(B, S, K))


# speedup vs baseline: 1.1936x; 1.1936x over previous
import functools

import jax
import jax.numpy as jnp
from jax.experimental import pallas as pl
from jax.experimental.pallas import tpu as pltpu

_T_R = 512
_T_B = 256


def _router_kernel(x_ref, wr_ref, probs_ref, idx_ref, w_ref, *, num_experts):
    xb = x_ref[...]
    logits = jnp.dot(xb, wr_ref[...], preferred_element_type=jnp.float32)
    m = jnp.max(logits, axis=-1, keepdims=True)
    p = jnp.exp(logits - m)
    probs = p / jnp.sum(p, axis=-1, keepdims=True)
    probs_ref[...] = probs
    iota = jax.lax.broadcasted_iota(jnp.int32, probs.shape, 1)
    m1 = jnp.max(probs, axis=-1, keepdims=True)
    i1 = jnp.min(jnp.where(probs == m1, iota, num_experts), axis=-1,
                 keepdims=True)
    masked = jnp.where(iota == i1, -1.0, probs)
    m2 = jnp.max(masked, axis=-1, keepdims=True)
    i2 = jnp.min(jnp.where(masked == m2, iota, num_experts), axis=-1,
                 keepdims=True)
    s = m1 + m2
    idx_ref[...] = jnp.concatenate([i1, i2], axis=-1)
    w_ref[...] = jnp.concatenate([m1 / s, m2 / s], axis=-1)


def _ffn_kernel(be_ref, first_ref, slot_ref, nxt_ref,
                x_ref, b1_ref, b2_ref, wg_ref, w1_hbm, w2_hbm, o_ref,
                w1_buf, w2_buf, s1, s2):
    g = pl.program_id(0)
    be = be_ref[g]
    slot = slot_ref[g]

    @pl.when(g == 0)
    def _():
        pltpu.make_async_copy(w1_hbm.at[be], w1_buf.at[slot],
                              s1.at[slot]).start()
        pltpu.make_async_copy(w2_hbm.at[be], w2_buf.at[slot],
                              s2.at[slot]).start()

    @pl.when(first_ref[g] == 1)
    def _():
        pltpu.make_async_copy(w1_hbm.at[be], w1_buf.at[slot],
                              s1.at[slot]).wait()
        pltpu.make_async_copy(w2_hbm.at[be], w2_buf.at[slot],
                              s2.at[slot]).wait()
        nxt = nxt_ref[g]

        @pl.when(nxt >= 0)
        def _():
            pltpu.make_async_copy(w1_hbm.at[nxt], w1_buf.at[1 - slot],
                                  s1.at[1 - slot]).start()
            pltpu.make_async_copy(w2_hbm.at[nxt], w2_buf.at[1 - slot],
                                  s2.at[1 - slot]).start()

    @pl.when(be >= 0)
    def _():
        xb = x_ref[...]
        h = jnp.dot(xb, w1_buf[slot], preferred_element_type=jnp.float32)
        h = h + b1_ref[0]
        a = 0.5 * h * (1.0 + jax.lax.erf(h * 0.7071067811865476))
        y = jnp.dot(a, w2_buf[slot], preferred_element_type=jnp.float32)
        y = y + b2_ref[0]
        o_ref[...] = y * wg_ref[...]


def kernel(x, Wr, W1, b1, W2, b2):
    B, S, D = x.shape
    E = Wr.shape[1]
    H = W1.shape[2]
    K = 2
    N = B * S
    P = N * K
    P_MAX = P + E * _T_B
    G_MAX = P_MAX // _T_B

    x2d = x.reshape(N, D)

    probs, sel, w = pl.pallas_call(
        functools.partial(_router_kernel, num_experts=E),
        grid=(N // _T_R,),
        in_specs=[
            pl.BlockSpec((_T_R, D), lambda i: (i, 0)),
            pl.BlockSpec((D, E), lambda i: (0, 0)),
        ],
        out_specs=[
            pl.BlockSpec((_T_R, E), lambda i: (i, 0)),
            pl.BlockSpec((_T_R, K), lambda i: (i, 0)),
            pl.BlockSpec((_T_R, K), lambda i: (i, 0)),
        ],
        out_shape=[
            jax.ShapeDtypeStruct((N, E), jnp.float32),
            jax.ShapeDtypeStruct((N, K), jnp.int32),
            jax.ShapeDtypeStruct((N, K), jnp.float32),
        ],
    )(x2d, Wr)

    e_flat = sel.reshape(P)
    onehot = (e_flat[:, None] == jnp.arange(E, dtype=jnp.int32)[None, :]
              ).astype(jnp.int32)
    ranks_all = jnp.cumsum(onehot, axis=0) - onehot
    rank = jnp.take_along_axis(ranks_all, e_flat[:, None], axis=1)[:, 0]
    counts = jnp.sum(onehot, axis=0)
    padded = ((counts + _T_B - 1) // _T_B) * _T_B
    ends = jnp.cumsum(padded)
    offs = ends - padded
    pos = offs[e_flat] + rank

    blk_start = jnp.arange(G_MAX, dtype=jnp.int32) * _T_B
    be = jnp.sum((blk_start[:, None] >= ends[None, :]).astype(jnp.int32),
                 axis=1)
    block_expert = jnp.where(be < E, be, -1).astype(jnp.int32)

    prev = jnp.concatenate(
        [jnp.full((1,), -2, jnp.int32), block_expert[:-1]])
    isfirst = ((block_expert != prev) & (block_expert >= 0)).astype(jnp.int32)
    slot = ((jnp.cumsum(isfirst) - 1) % 2).astype(jnp.int32)
    gi = jnp.arange(G_MAX, dtype=jnp.int32)
    start_pos = jnp.where(isfirst == 1, gi, G_MAX)
    nxt_start = jnp.flip(jax.lax.cummin(jnp.flip(
        jnp.concatenate([start_pos[1:], jnp.full((1,), G_MAX, jnp.int32)]))))
    nxt_e = jnp.where(nxt_start < G_MAX,
                      block_expert[jnp.minimum(nxt_start, G_MAX - 1)], -1)

    row_token = jnp.zeros((P_MAX,), jnp.int32).at[pos].set(
        jnp.arange(P, dtype=jnp.int32) // K)
    wg = jnp.zeros((P_MAX,), jnp.float32).at[pos].set(w.reshape(P))

    xg = x2d[row_token]

    grid_spec = pltpu.PrefetchScalarGridSpec(
        num_scalar_prefetch=4,
        grid=(G_MAX,),
        in_specs=[
            pl.BlockSpec((_T_B, D), lambda g, be, f, s, n: (g, 0)),
            pl.BlockSpec((1, 1, H),
                         lambda g, be, f, s, n: (jnp.maximum(be[g], 0), 0, 0)),
            pl.BlockSpec((1, 1, D),
                         lambda g, be, f, s, n: (jnp.maximum(be[g], 0), 0, 0)),
            pl.BlockSpec((_T_B, 1), lambda g, be, f, s, n: (g, 0)),
            pl.BlockSpec(memory_space=pl.ANY),
            pl.BlockSpec(memory_space=pl.ANY),
        ],
        out_specs=pl.BlockSpec((_T_B, D), lambda g, be, f, s, n: (g, 0)),
        scratch_shapes=[
            pltpu.VMEM((2, D, H), jnp.float32),
            pltpu.VMEM((2, H, D), jnp.float32),
            pltpu.SemaphoreType.DMA((2,)),
            pltpu.SemaphoreType.DMA((2,)),
        ],
    )
    yg = pl.pallas_call(
        _ffn_kernel,
        grid_spec=grid_spec,
        out_shape=jax.ShapeDtypeStruct((P_MAX, D), jnp.float32),
    )(block_expert, isfirst, slot, nxt_e,
      xg, b1.reshape(E, 1, H), b2.reshape(E, 1, D), wg[:, None], W1, W2)

    pos2 = pos.reshape(N, K)
    out2d = yg[pos2[:, 0]] + yg[pos2[:, 1]]

    return (out2d.reshape(B, S, D), probs.reshape(B, S, E),
            sel.reshape(B, S, K), w.reshape(B, S, K))


# speedup vs baseline: 1.4200x; 1.1897x over previous
import functools

import jax
import jax.numpy as jnp
from jax import lax
from jax.experimental import pallas as pl
from jax.experimental.pallas import tpu as pltpu
from jax.experimental.pallas import tpu_sc as plsc

_T_R = 512
_T_B = 256


def _router_kernel(x_ref, wr_ref, probs_ref, idx_ref, w_ref, *, num_experts):
    xb = x_ref[...]
    logits = jnp.dot(xb, wr_ref[...], preferred_element_type=jnp.float32)
    m = jnp.max(logits, axis=-1, keepdims=True)
    p = jnp.exp(logits - m)
    probs = p / jnp.sum(p, axis=-1, keepdims=True)
    probs_ref[...] = probs
    iota = jax.lax.broadcasted_iota(jnp.int32, probs.shape, 1)
    m1 = jnp.max(probs, axis=-1, keepdims=True)
    i1 = jnp.min(jnp.where(probs == m1, iota, num_experts), axis=-1,
                 keepdims=True)
    masked = jnp.where(iota == i1, -1.0, probs)
    m2 = jnp.max(masked, axis=-1, keepdims=True)
    i2 = jnp.min(jnp.where(masked == m2, iota, num_experts), axis=-1,
                 keepdims=True)
    s = m1 + m2
    idx_ref[...] = jnp.concatenate([i1, i2], axis=-1)
    w_ref[...] = jnp.concatenate([m1 / s, m2 / s], axis=-1)


def _ffn_kernel(be_ref, first_ref, slot_ref, nxt_ref,
                x_ref, b1_ref, b2_ref, wg_ref, w1_hbm, w2_hbm, o_ref,
                w1_buf, w2_buf, s1, s2):
    g = pl.program_id(0)
    be = be_ref[g]
    slot = slot_ref[g]

    @pl.when(g == 0)
    def _():
        pltpu.make_async_copy(w1_hbm.at[be], w1_buf.at[slot],
                              s1.at[slot]).start()
        pltpu.make_async_copy(w2_hbm.at[be], w2_buf.at[slot],
                              s2.at[slot]).start()

    @pl.when(first_ref[g] == 1)
    def _():
        pltpu.make_async_copy(w1_hbm.at[be], w1_buf.at[slot],
                              s1.at[slot]).wait()
        pltpu.make_async_copy(w2_hbm.at[be], w2_buf.at[slot],
                              s2.at[slot]).wait()
        nxt = nxt_ref[g]

        @pl.when(nxt >= 0)
        def _():
            pltpu.make_async_copy(w1_hbm.at[nxt], w1_buf.at[1 - slot],
                                  s1.at[1 - slot]).start()
            pltpu.make_async_copy(w2_hbm.at[nxt], w2_buf.at[1 - slot],
                                  s2.at[1 - slot]).start()

    @pl.when(be >= 0)
    def _():
        xb = x_ref[...]
        h = jnp.dot(xb, w1_buf[slot], preferred_element_type=jnp.float32)
        h = h + b1_ref[0]
        a = 0.5 * h * (1.0 + jax.lax.erf(h * 0.7071067811865476))
        y = jnp.dot(a, w2_buf[slot], preferred_element_type=jnp.float32)
        y = y + b2_ref[0]
        o_ref[...] = y * wg_ref[...]


def kernel(x, Wr, W1, b1, W2, b2):
    B, S, D = x.shape
    E = Wr.shape[1]
    H = W1.shape[2]
    K = 2
    N = B * S
    P = N * K
    P_MAX = P + E * _T_B
    G_MAX = P_MAX // _T_B

    x2d = x.reshape(N, D)

    probs, sel, w = pl.pallas_call(
        functools.partial(_router_kernel, num_experts=E),
        grid=(N // _T_R,),
        in_specs=[
            pl.BlockSpec((_T_R, D), lambda i: (i, 0)),
            pl.BlockSpec((D, E), lambda i: (0, 0)),
        ],
        out_specs=[
            pl.BlockSpec((_T_R, E), lambda i: (i, 0)),
            pl.BlockSpec((_T_R, K), lambda i: (i, 0)),
            pl.BlockSpec((_T_R, K), lambda i: (i, 0)),
        ],
        out_shape=[
            jax.ShapeDtypeStruct((N, E), jnp.float32),
            jax.ShapeDtypeStruct((N, K), jnp.int32),
            jax.ShapeDtypeStruct((N, K), jnp.float32),
        ],
    )(x2d, Wr)

    e_flat = sel.reshape(P)
    onehot = (e_flat[:, None] == jnp.arange(E, dtype=jnp.int32)[None, :]
              ).astype(jnp.int32)
    ranks_all = jnp.cumsum(onehot, axis=0) - onehot
    rank = jnp.take_along_axis(ranks_all, e_flat[:, None], axis=1)[:, 0]
    counts = jnp.sum(onehot, axis=0)
    padded = ((counts + _T_B - 1) // _T_B) * _T_B
    ends = jnp.cumsum(padded)
    offs = ends - padded
    pos = offs[e_flat] + rank

    blk_start = jnp.arange(G_MAX, dtype=jnp.int32) * _T_B
    be = jnp.sum((blk_start[:, None] >= ends[None, :]).astype(jnp.int32),
                 axis=1)
    block_expert = jnp.where(be < E, be, -1).astype(jnp.int32)

    prev = jnp.concatenate(
        [jnp.full((1,), -2, jnp.int32), block_expert[:-1]])
    isfirst = ((block_expert != prev) & (block_expert >= 0)).astype(jnp.int32)
    slot = ((jnp.cumsum(isfirst) - 1) % 2).astype(jnp.int32)
    gi = jnp.arange(G_MAX, dtype=jnp.int32)
    start_pos = jnp.where(isfirst == 1, gi, G_MAX)
    nxt_start = jnp.flip(jax.lax.cummin(jnp.flip(
        jnp.concatenate([start_pos[1:], jnp.full((1,), G_MAX, jnp.int32)]))))
    nxt_e = jnp.where(nxt_start < G_MAX,
                      block_expert[jnp.minimum(nxt_start, G_MAX - 1)], -1)

    pos2 = pos.reshape(N, K)
    pos0 = pos2[:, 0]
    pos1 = pos2[:, 1]
    w0 = w[:, 0]
    w1 = w[:, 1]

    NW = 32
    TPW = N // NW
    mesh = plsc.VectorSubcoreMesh(core_axis_name="c", subcore_axis_name="s")

    def _dispatch_kernel(x_hbm, p0_hbm, p1_hbm, w0_hbm, w1_hbm,
                         xg_hbm, wg_hbm, rows_v, idx_v, val_v, sem):
        wid = lax.axis_index("s") * 2 + lax.axis_index("c")
        base = wid * TPW
        pltpu.sync_copy(x_hbm.at[pl.ds(base, TPW)], rows_v)
        pltpu.sync_copy(p0_hbm.at[pl.ds(base, TPW)], idx_v)
        pltpu.async_copy(rows_v, xg_hbm.at[idx_v], sem).wait()
        pltpu.sync_copy(w0_hbm.at[pl.ds(base, TPW)], val_v)
        pltpu.async_copy(val_v, wg_hbm.at[idx_v], sem).wait()
        pltpu.sync_copy(p1_hbm.at[pl.ds(base, TPW)], idx_v)
        pltpu.async_copy(rows_v, xg_hbm.at[idx_v], sem).wait()
        pltpu.sync_copy(w1_hbm.at[pl.ds(base, TPW)], val_v)
        pltpu.async_copy(val_v, wg_hbm.at[idx_v], sem).wait()

    xg, wg = pl.kernel(
        _dispatch_kernel,
        mesh=mesh,
        out_type=[
            jax.ShapeDtypeStruct((P_MAX, D), jnp.float32),
            jax.ShapeDtypeStruct((P_MAX,), jnp.float32),
        ],
        scratch_types=[
            pltpu.VMEM((TPW, D), jnp.float32),
            pltpu.VMEM((TPW,), jnp.int32),
            pltpu.VMEM((TPW,), jnp.float32),
            pltpu.SemaphoreType.DMA,
        ],
    )(x2d, pos0, pos1, w0, w1)

    grid_spec = pltpu.PrefetchScalarGridSpec(
        num_scalar_prefetch=4,
        grid=(G_MAX,),
        in_specs=[
            pl.BlockSpec((_T_B, D), lambda g, be, f, s, n: (g, 0)),
            pl.BlockSpec((1, 1, H),
                         lambda g, be, f, s, n: (jnp.maximum(be[g], 0), 0, 0)),
            pl.BlockSpec((1, 1, D),
                         lambda g, be, f, s, n: (jnp.maximum(be[g], 0), 0, 0)),
            pl.BlockSpec((_T_B, 1), lambda g, be, f, s, n: (g, 0)),
            pl.BlockSpec(memory_space=pl.ANY),
            pl.BlockSpec(memory_space=pl.ANY),
        ],
        out_specs=pl.BlockSpec((_T_B, D), lambda g, be, f, s, n: (g, 0)),
        scratch_shapes=[
            pltpu.VMEM((2, D, H), jnp.float32),
            pltpu.VMEM((2, H, D), jnp.float32),
            pltpu.SemaphoreType.DMA((2,)),
            pltpu.SemaphoreType.DMA((2,)),
        ],
    )
    yg = pl.pallas_call(
        _ffn_kernel,
        grid_spec=grid_spec,
        out_shape=jax.ShapeDtypeStruct((P_MAX, D), jnp.float32),
    )(block_expert, isfirst, slot, nxt_e,
      xg, b1.reshape(E, 1, H), b2.reshape(E, 1, D), wg[:, None], W1, W2)

    out2d = yg[pos0] + yg[pos1]

    return (out2d.reshape(B, S, D), probs.reshape(B, S, E),
            sel.reshape(B, S, K), w.reshape(B, S, K))


# speedup vs baseline: 1.5700x; 1.1056x over previous
import functools

import jax
import jax.numpy as jnp
from jax import lax
from jax.experimental import pallas as pl
from jax.experimental.pallas import tpu as pltpu
from jax.experimental.pallas import tpu_sc as plsc

_T_R = 512
_T_B = 256


def _router_kernel(x_ref, wr_ref, probs_ref, idx_ref, w_ref, *, num_experts):
    xb = x_ref[...]
    logits = jnp.dot(xb, wr_ref[...], preferred_element_type=jnp.float32)
    m = jnp.max(logits, axis=-1, keepdims=True)
    p = jnp.exp(logits - m)
    probs = p / jnp.sum(p, axis=-1, keepdims=True)
    probs_ref[...] = probs
    iota = jax.lax.broadcasted_iota(jnp.int32, probs.shape, 1)
    m1 = jnp.max(probs, axis=-1, keepdims=True)
    i1 = jnp.min(jnp.where(probs == m1, iota, num_experts), axis=-1,
                 keepdims=True)
    masked = jnp.where(iota == i1, -1.0, probs)
    m2 = jnp.max(masked, axis=-1, keepdims=True)
    i2 = jnp.min(jnp.where(masked == m2, iota, num_experts), axis=-1,
                 keepdims=True)
    s = m1 + m2
    idx_ref[...] = jnp.concatenate([i1, i2], axis=-1)
    w_ref[...] = jnp.concatenate([m1 / s, m2 / s], axis=-1)


def _ffn_kernel(be_ref, first_ref, slot_ref, nxt_ref,
                x_ref, b1_ref, b2_ref, wg_ref, w1_hbm, w2_hbm, o_ref,
                w1_buf, w2_buf, s1, s2):
    g = pl.program_id(0)
    be = be_ref[g]
    slot = slot_ref[g]

    @pl.when(g == 0)
    def _():
        pltpu.make_async_copy(w1_hbm.at[be], w1_buf.at[slot],
                              s1.at[slot]).start()
        pltpu.make_async_copy(w2_hbm.at[be], w2_buf.at[slot],
                              s2.at[slot]).start()

    @pl.when(first_ref[g] == 1)
    def _():
        pltpu.make_async_copy(w1_hbm.at[be], w1_buf.at[slot],
                              s1.at[slot]).wait()
        pltpu.make_async_copy(w2_hbm.at[be], w2_buf.at[slot],
                              s2.at[slot]).wait()
        nxt = nxt_ref[g]

        @pl.when(nxt >= 0)
        def _():
            pltpu.make_async_copy(w1_hbm.at[nxt], w1_buf.at[1 - slot],
                                  s1.at[1 - slot]).start()
            pltpu.make_async_copy(w2_hbm.at[nxt], w2_buf.at[1 - slot],
                                  s2.at[1 - slot]).start()

    @pl.when(be >= 0)
    def _():
        xb = x_ref[...]
        h = jnp.dot(xb, w1_buf[slot], preferred_element_type=jnp.float32)
        h = h + b1_ref[0]
        a = 0.5 * h * (1.0 + jax.lax.erf(h * 0.7071067811865476))
        y = jnp.dot(a, w2_buf[slot], preferred_element_type=jnp.float32)
        y = y + b2_ref[0]
        o_ref[...] = y * wg_ref[...]


def kernel(x, Wr, W1, b1, W2, b2):
    B, S, D = x.shape
    E = Wr.shape[1]
    H = W1.shape[2]
    K = 2
    N = B * S
    P = N * K
    P_MAX = P + E * _T_B
    G_MAX = P_MAX // _T_B

    x2d = x.reshape(N, D)

    probs, sel, w = pl.pallas_call(
        functools.partial(_router_kernel, num_experts=E),
        grid=(N // _T_R,),
        in_specs=[
            pl.BlockSpec((_T_R, D), lambda i: (i, 0)),
            pl.BlockSpec((D, E), lambda i: (0, 0)),
        ],
        out_specs=[
            pl.BlockSpec((_T_R, E), lambda i: (i, 0)),
            pl.BlockSpec((_T_R, K), lambda i: (i, 0)),
            pl.BlockSpec((_T_R, K), lambda i: (i, 0)),
        ],
        out_shape=[
            jax.ShapeDtypeStruct((N, E), jnp.float32),
            jax.ShapeDtypeStruct((N, K), jnp.int32),
            jax.ShapeDtypeStruct((N, K), jnp.float32),
        ],
    )(x2d, Wr)

    e_flat = sel.reshape(P)
    onehot = (e_flat[:, None] == jnp.arange(E, dtype=jnp.int32)[None, :]
              ).astype(jnp.int32)
    ranks_all = jnp.cumsum(onehot, axis=0) - onehot
    rank = jnp.take_along_axis(ranks_all, e_flat[:, None], axis=1)[:, 0]
    counts = jnp.sum(onehot, axis=0)
    padded = ((counts + _T_B - 1) // _T_B) * _T_B
    ends = jnp.cumsum(padded)
    offs = ends - padded
    pos = offs[e_flat] + rank

    blk_start = jnp.arange(G_MAX, dtype=jnp.int32) * _T_B
    be = jnp.sum((blk_start[:, None] >= ends[None, :]).astype(jnp.int32),
                 axis=1)
    block_expert = jnp.where(be < E, be, -1).astype(jnp.int32)

    prev = jnp.concatenate(
        [jnp.full((1,), -2, jnp.int32), block_expert[:-1]])
    isfirst = ((block_expert != prev) & (block_expert >= 0)).astype(jnp.int32)
    slot = ((jnp.cumsum(isfirst) - 1) % 2).astype(jnp.int32)
    gi = jnp.arange(G_MAX, dtype=jnp.int32)
    start_pos = jnp.where(isfirst == 1, gi, G_MAX)
    nxt_start = jnp.flip(jax.lax.cummin(jnp.flip(
        jnp.concatenate([start_pos[1:], jnp.full((1,), G_MAX, jnp.int32)]))))
    nxt_e = jnp.where(nxt_start < G_MAX,
                      block_expert[jnp.minimum(nxt_start, G_MAX - 1)], -1)

    pos2 = pos.reshape(N, K)
    pos0 = pos2[:, 0]
    pos1 = pos2[:, 1]
    w0 = w[:, 0]
    w1 = w[:, 1]

    NW = 32
    TPW = N // NW
    mesh = plsc.VectorSubcoreMesh(core_axis_name="c", subcore_axis_name="s")

    def _dispatch_kernel(x_hbm, p0_hbm, p1_hbm, w0_hbm, w1_hbm,
                         xg_hbm, wg_hbm, rows_v, idx0_v, idx1_v,
                         val0_v, val1_v, sem0, sem1, sem2, sem3):
        wid = lax.axis_index("s") * 2 + lax.axis_index("c")
        base = wid * TPW
        pltpu.sync_copy(p0_hbm.at[pl.ds(base, TPW)], idx0_v)
        pltpu.sync_copy(p1_hbm.at[pl.ds(base, TPW)], idx1_v)
        pltpu.sync_copy(w0_hbm.at[pl.ds(base, TPW)], val0_v)
        pltpu.sync_copy(w1_hbm.at[pl.ds(base, TPW)], val1_v)
        pltpu.sync_copy(x_hbm.at[pl.ds(base, TPW)], rows_v)
        c0 = pltpu.async_copy(rows_v, xg_hbm.at[idx0_v], sem0)
        c1 = pltpu.async_copy(rows_v, xg_hbm.at[idx1_v], sem1)
        c2 = pltpu.async_copy(val0_v, wg_hbm.at[idx0_v], sem2)
        c3 = pltpu.async_copy(val1_v, wg_hbm.at[idx1_v], sem3)
        c0.wait()
        c1.wait()
        c2.wait()
        c3.wait()

    xg, wg = pl.kernel(
        _dispatch_kernel,
        mesh=mesh,
        out_type=[
            jax.ShapeDtypeStruct((P_MAX, D), jnp.float32),
            jax.ShapeDtypeStruct((P_MAX,), jnp.float32),
        ],
        scratch_types=[
            pltpu.VMEM((TPW, D), jnp.float32),
            pltpu.VMEM((TPW,), jnp.int32),
            pltpu.VMEM((TPW,), jnp.int32),
            pltpu.VMEM((TPW,), jnp.float32),
            pltpu.VMEM((TPW,), jnp.float32),
            pltpu.SemaphoreType.DMA,
            pltpu.SemaphoreType.DMA,
            pltpu.SemaphoreType.DMA,
            pltpu.SemaphoreType.DMA,
        ],
    )(x2d, pos0, pos1, w0, w1)

    grid_spec = pltpu.PrefetchScalarGridSpec(
        num_scalar_prefetch=4,
        grid=(G_MAX,),
        in_specs=[
            pl.BlockSpec((_T_B, D), lambda g, be, f, s, n: (g, 0)),
            pl.BlockSpec((1, 1, H),
                         lambda g, be, f, s, n: (jnp.maximum(be[g], 0), 0, 0)),
            pl.BlockSpec((1, 1, D),
                         lambda g, be, f, s, n: (jnp.maximum(be[g], 0), 0, 0)),
            pl.BlockSpec((_T_B, 1), lambda g, be, f, s, n: (g, 0)),
            pl.BlockSpec(memory_space=pl.ANY),
            pl.BlockSpec(memory_space=pl.ANY),
        ],
        out_specs=pl.BlockSpec((_T_B, D), lambda g, be, f, s, n: (g, 0)),
        scratch_shapes=[
            pltpu.VMEM((2, D, H), jnp.float32),
            pltpu.VMEM((2, H, D), jnp.float32),
            pltpu.SemaphoreType.DMA((2,)),
            pltpu.SemaphoreType.DMA((2,)),
        ],
    )
    yg = pl.pallas_call(
        _ffn_kernel,
        grid_spec=grid_spec,
        out_shape=jax.ShapeDtypeStruct((P_MAX, D), jnp.float32),
    )(block_expert, isfirst, slot, nxt_e,
      xg, b1.reshape(E, 1, H), b2.reshape(E, 1, D), wg[:, None], W1, W2)

    SUB = 2
    TPS = TPW // SUB
    CHUNK_V = TPS * D // 16

    def _combine_kernel(yg_hbm, p0_hbm, p1_hbm, o_hbm,
                        i0_v, i1_v, b0_v, b1_v, csem0, csem1):
        wid = lax.axis_index("s") * 2 + lax.axis_index("c")
        for sub in range(SUB):
            base = wid * TPW + sub * TPS
            pltpu.sync_copy(p0_hbm.at[pl.ds(base, TPS)], i0_v)
            pltpu.sync_copy(p1_hbm.at[pl.ds(base, TPS)], i1_v)
            g0 = pltpu.async_copy(yg_hbm.at[i0_v], b0_v, csem0)
            g1 = pltpu.async_copy(yg_hbm.at[i1_v], b1_v, csem1)
            g0.wait()
            g1.wait()

            def _add(i, _):
                r = i // (D // 16)
                c = (i - r * (D // 16)) * 16
                b0_v[r, pl.ds(c, 16)] = (b0_v[r, pl.ds(c, 16)]
                                         + b1_v[r, pl.ds(c, 16)])
                return 0

            lax.fori_loop(0, CHUNK_V, _add, 0, unroll=8)
            pltpu.sync_copy(b0_v, o_hbm.at[pl.ds(base, TPS)])

    out2d = pl.kernel(
        _combine_kernel,
        mesh=mesh,
        out_type=jax.ShapeDtypeStruct((N, D), jnp.float32),
        scratch_types=[
            pltpu.VMEM((TPS,), jnp.int32),
            pltpu.VMEM((TPS,), jnp.int32),
            pltpu.VMEM((TPS, D), jnp.float32),
            pltpu.VMEM((TPS, D), jnp.float32),
            pltpu.SemaphoreType.DMA,
            pltpu.SemaphoreType.DMA,
        ],
    )(yg, pos0, pos1)

    return (out2d.reshape(B, S, D), probs.reshape(B, S, E),
            sel.reshape(B, S, K), w.reshape(B, S, K))


# speedup vs baseline: 1.6092x; 1.0250x over previous
import functools

import jax
import jax.numpy as jnp
from jax import lax
from jax.experimental import pallas as pl
from jax.experimental.pallas import tpu as pltpu
from jax.experimental.pallas import tpu_sc as plsc

_T_R = 512
_T_B = 512


def _router_kernel(x_ref, wr_ref, probs_ref, idx_ref, w_ref, *, num_experts):
    xb = x_ref[...]
    logits = jnp.dot(xb, wr_ref[...], preferred_element_type=jnp.float32)
    m = jnp.max(logits, axis=-1, keepdims=True)
    p = jnp.exp(logits - m)
    probs = p / jnp.sum(p, axis=-1, keepdims=True)
    probs_ref[...] = probs
    iota = jax.lax.broadcasted_iota(jnp.int32, probs.shape, 1)
    m1 = jnp.max(probs, axis=-1, keepdims=True)
    i1 = jnp.min(jnp.where(probs == m1, iota, num_experts), axis=-1,
                 keepdims=True)
    masked = jnp.where(iota == i1, -1.0, probs)
    m2 = jnp.max(masked, axis=-1, keepdims=True)
    i2 = jnp.min(jnp.where(masked == m2, iota, num_experts), axis=-1,
                 keepdims=True)
    s = m1 + m2
    idx_ref[...] = jnp.concatenate([i1, i2], axis=-1)
    w_ref[...] = jnp.concatenate([m1 / s, m2 / s], axis=-1)


def _ffn_kernel(be_ref, first_ref, slot_ref, nxt_ref,
                x_ref, b1_ref, b2_ref, wg_ref, w1_hbm, w2_hbm, o_ref,
                w1_buf, w2_buf, s1, s2):
    g = pl.program_id(0)
    be = be_ref[g]
    slot = slot_ref[g]

    @pl.when(g == 0)
    def _():
        pltpu.make_async_copy(w1_hbm.at[be], w1_buf.at[slot],
                              s1.at[slot]).start()
        pltpu.make_async_copy(w2_hbm.at[be], w2_buf.at[slot],
                              s2.at[slot]).start()

    @pl.when(first_ref[g] == 1)
    def _():
        pltpu.make_async_copy(w1_hbm.at[be], w1_buf.at[slot],
                              s1.at[slot]).wait()
        pltpu.make_async_copy(w2_hbm.at[be], w2_buf.at[slot],
                              s2.at[slot]).wait()
        nxt = nxt_ref[g]

        @pl.when(nxt >= 0)
        def _():
            pltpu.make_async_copy(w1_hbm.at[nxt], w1_buf.at[1 - slot],
                                  s1.at[1 - slot]).start()
            pltpu.make_async_copy(w2_hbm.at[nxt], w2_buf.at[1 - slot],
                                  s2.at[1 - slot]).start()

    @pl.when(be >= 0)
    def _():
        xb = x_ref[...]
        h = jnp.dot(xb, w1_buf[slot], preferred_element_type=jnp.float32)
        h = h + b1_ref[0]
        a = 0.5 * h * (1.0 + jax.lax.erf(h * 0.7071067811865476))
        y = jnp.dot(a, w2_buf[slot], preferred_element_type=jnp.float32)
        y = y + b2_ref[0]
        o_ref[...] = y * wg_ref[...]


def kernel(x, Wr, W1, b1, W2, b2):
    B, S, D = x.shape
    E = Wr.shape[1]
    H = W1.shape[2]
    K = 2
    N = B * S
    P = N * K
    P_MAX = P + E * _T_B
    G_MAX = P_MAX // _T_B

    x2d = x.reshape(N, D)

    probs, sel, w = pl.pallas_call(
        functools.partial(_router_kernel, num_experts=E),
        grid=(N // _T_R,),
        in_specs=[
            pl.BlockSpec((_T_R, D), lambda i: (i, 0)),
            pl.BlockSpec((D, E), lambda i: (0, 0)),
        ],
        out_specs=[
            pl.BlockSpec((_T_R, E), lambda i: (i, 0)),
            pl.BlockSpec((_T_R, K), lambda i: (i, 0)),
            pl.BlockSpec((_T_R, K), lambda i: (i, 0)),
        ],
        out_shape=[
            jax.ShapeDtypeStruct((N, E), jnp.float32),
            jax.ShapeDtypeStruct((N, K), jnp.int32),
            jax.ShapeDtypeStruct((N, K), jnp.float32),
        ],
    )(x2d, Wr)

    e_flat = sel.reshape(P)
    onehot = (e_flat[:, None] == jnp.arange(E, dtype=jnp.int32)[None, :]
              ).astype(jnp.int32)
    ranks_all = jnp.cumsum(onehot, axis=0) - onehot
    rank = jnp.take_along_axis(ranks_all, e_flat[:, None], axis=1)[:, 0]
    counts = jnp.sum(onehot, axis=0)
    padded = ((counts + _T_B - 1) // _T_B) * _T_B
    ends = jnp.cumsum(padded)
    offs = ends - padded
    pos = offs[e_flat] + rank

    blk_start = jnp.arange(G_MAX, dtype=jnp.int32) * _T_B
    be = jnp.sum((blk_start[:, None] >= ends[None, :]).astype(jnp.int32),
                 axis=1)
    block_expert = jnp.where(be < E, be, -1).astype(jnp.int32)

    prev = jnp.concatenate(
        [jnp.full((1,), -2, jnp.int32), block_expert[:-1]])
    isfirst = ((block_expert != prev) & (block_expert >= 0)).astype(jnp.int32)
    slot = ((jnp.cumsum(isfirst) - 1) % 2).astype(jnp.int32)
    gi = jnp.arange(G_MAX, dtype=jnp.int32)
    start_pos = jnp.where(isfirst == 1, gi, G_MAX)
    nxt_start = jnp.flip(jax.lax.cummin(jnp.flip(
        jnp.concatenate([start_pos[1:], jnp.full((1,), G_MAX, jnp.int32)]))))
    nxt_e = jnp.where(nxt_start < G_MAX,
                      block_expert[jnp.minimum(nxt_start, G_MAX - 1)], -1)

    pos2 = pos.reshape(N, K)
    pos0 = pos2[:, 0]
    pos1 = pos2[:, 1]
    w0 = w[:, 0]
    w1 = w[:, 1]

    NW = 32
    TPW = N // NW
    mesh = plsc.VectorSubcoreMesh(core_axis_name="c", subcore_axis_name="s")

    def _dispatch_kernel(x_hbm, p0_hbm, p1_hbm, w0_hbm, w1_hbm,
                         xg_hbm, wg_hbm, rows_v, idx0_v, idx1_v,
                         val0_v, val1_v, sem0, sem1, sem2, sem3):
        wid = lax.axis_index("s") * 2 + lax.axis_index("c")
        base = wid * TPW
        pltpu.sync_copy(p0_hbm.at[pl.ds(base, TPW)], idx0_v)
        pltpu.sync_copy(p1_hbm.at[pl.ds(base, TPW)], idx1_v)
        pltpu.sync_copy(w0_hbm.at[pl.ds(base, TPW)], val0_v)
        pltpu.sync_copy(w1_hbm.at[pl.ds(base, TPW)], val1_v)
        pltpu.sync_copy(x_hbm.at[pl.ds(base, TPW)], rows_v)
        c0 = pltpu.async_copy(rows_v, xg_hbm.at[idx0_v], sem0)
        c1 = pltpu.async_copy(rows_v, xg_hbm.at[idx1_v], sem1)
        c2 = pltpu.async_copy(val0_v, wg_hbm.at[idx0_v], sem2)
        c3 = pltpu.async_copy(val1_v, wg_hbm.at[idx1_v], sem3)
        c0.wait()
        c1.wait()
        c2.wait()
        c3.wait()

    xg, wg = pl.kernel(
        _dispatch_kernel,
        mesh=mesh,
        out_type=[
            jax.ShapeDtypeStruct((P_MAX, D), jnp.float32),
            jax.ShapeDtypeStruct((P_MAX,), jnp.float32),
        ],
        scratch_types=[
            pltpu.VMEM((TPW, D), jnp.float32),
            pltpu.VMEM((TPW,), jnp.int32),
            pltpu.VMEM((TPW,), jnp.int32),
            pltpu.VMEM((TPW,), jnp.float32),
            pltpu.VMEM((TPW,), jnp.float32),
            pltpu.SemaphoreType.DMA,
            pltpu.SemaphoreType.DMA,
            pltpu.SemaphoreType.DMA,
            pltpu.SemaphoreType.DMA,
        ],
    )(x2d, pos0, pos1, w0, w1)

    grid_spec = pltpu.PrefetchScalarGridSpec(
        num_scalar_prefetch=4,
        grid=(G_MAX,),
        in_specs=[
            pl.BlockSpec((_T_B, D), lambda g, be, f, s, n: (g, 0)),
            pl.BlockSpec((1, 1, H),
                         lambda g, be, f, s, n: (jnp.maximum(be[g], 0), 0, 0)),
            pl.BlockSpec((1, 1, D),
                         lambda g, be, f, s, n: (jnp.maximum(be[g], 0), 0, 0)),
            pl.BlockSpec((_T_B, 1), lambda g, be, f, s, n: (g, 0)),
            pl.BlockSpec(memory_space=pl.ANY),
            pl.BlockSpec(memory_space=pl.ANY),
        ],
        out_specs=pl.BlockSpec((_T_B, D), lambda g, be, f, s, n: (g, 0)),
        scratch_shapes=[
            pltpu.VMEM((2, D, H), jnp.float32),
            pltpu.VMEM((2, H, D), jnp.float32),
            pltpu.SemaphoreType.DMA((2,)),
            pltpu.SemaphoreType.DMA((2,)),
        ],
    )
    yg = pl.pallas_call(
        _ffn_kernel,
        grid_spec=grid_spec,
        out_shape=jax.ShapeDtypeStruct((P_MAX, D), jnp.float32),
    )(block_expert, isfirst, slot, nxt_e,
      xg, b1.reshape(E, 1, H), b2.reshape(E, 1, D), wg[:, None], W1, W2)

    SUB = 2
    TPS = TPW // SUB
    CHUNK_V = TPS * D // 16

    def _combine_kernel(yg_hbm, p0_hbm, p1_hbm, o_hbm,
                        i0_v, i1_v, b0_v, b1_v, csem0, csem1):
        wid = lax.axis_index("s") * 2 + lax.axis_index("c")
        for sub in range(SUB):
            base = wid * TPW + sub * TPS
            pltpu.sync_copy(p0_hbm.at[pl.ds(base, TPS)], i0_v)
            pltpu.sync_copy(p1_hbm.at[pl.ds(base, TPS)], i1_v)
            g0 = pltpu.async_copy(yg_hbm.at[i0_v], b0_v, csem0)
            g1 = pltpu.async_copy(yg_hbm.at[i1_v], b1_v, csem1)
            g0.wait()
            g1.wait()

            def _add(i, _):
                r = i // (D // 16)
                c = (i - r * (D // 16)) * 16
                b0_v[r, pl.ds(c, 16)] = (b0_v[r, pl.ds(c, 16)]
                                         + b1_v[r, pl.ds(c, 16)])
                return 0

            lax.fori_loop(0, CHUNK_V, _add, 0, unroll=8)
            pltpu.sync_copy(b0_v, o_hbm.at[pl.ds(base, TPS)])

    out2d = pl.kernel(
        _combine_kernel,
        mesh=mesh,
        out_type=jax.ShapeDtypeStruct((N, D), jnp.float32),
        scratch_types=[
            pltpu.VMEM((TPS,), jnp.int32),
            pltpu.VMEM((TPS,), jnp.int32),
            pltpu.VMEM((TPS, D), jnp.float32),
            pltpu.VMEM((TPS, D), jnp.float32),
            pltpu.SemaphoreType.DMA,
            pltpu.SemaphoreType.DMA,
        ],
    )(yg, pos0, pos1)

    return (out2d.reshape(B, S, D), probs.reshape(B, S, E),
            sel.reshape(B, S, K), w.reshape(B, S, K))


# speedup vs baseline: 1.6296x; 1.0127x over previous
import functools

import jax
import jax.numpy as jnp
from jax import lax
from jax.experimental import pallas as pl
from jax.experimental.pallas import tpu as pltpu
from jax.experimental.pallas import tpu_sc as plsc

_T_R = 512
_T_B = 512


def _router_kernel(x_ref, wr_ref, probs_ref, idx_ref, w_ref, *, num_experts):
    xb = x_ref[...]
    logits = jnp.dot(xb, wr_ref[...], preferred_element_type=jnp.float32)
    m = jnp.max(logits, axis=-1, keepdims=True)
    p = jnp.exp(logits - m)
    probs = p / jnp.sum(p, axis=-1, keepdims=True)
    probs_ref[...] = probs
    iota = jax.lax.broadcasted_iota(jnp.int32, probs.shape, 1)
    m1 = jnp.max(probs, axis=-1, keepdims=True)
    i1 = jnp.min(jnp.where(probs == m1, iota, num_experts), axis=-1,
                 keepdims=True)
    masked = jnp.where(iota == i1, -1.0, probs)
    m2 = jnp.max(masked, axis=-1, keepdims=True)
    i2 = jnp.min(jnp.where(masked == m2, iota, num_experts), axis=-1,
                 keepdims=True)
    s = m1 + m2
    idx_ref[...] = jnp.concatenate([i1, i2], axis=-1)
    w_ref[...] = jnp.concatenate([m1 / s, m2 / s], axis=-1)


def _ffn_kernel(be_ref, first_ref, slot_ref, nxt_ref,
                x_ref, b1_ref, b2_ref, wg_ref, w1_hbm, w2_hbm, o_ref,
                w1_buf, w2_buf, s1, s2):
    g = pl.program_id(0)
    be = be_ref[g]
    slot = slot_ref[g]

    @pl.when(g == 0)
    def _():
        pltpu.make_async_copy(w1_hbm.at[be], w1_buf.at[slot],
                              s1.at[slot]).start()
        pltpu.make_async_copy(w2_hbm.at[be], w2_buf.at[slot],
                              s2.at[slot]).start()

    @pl.when(first_ref[g] == 1)
    def _():
        pltpu.make_async_copy(w1_hbm.at[be], w1_buf.at[slot],
                              s1.at[slot]).wait()
        pltpu.make_async_copy(w2_hbm.at[be], w2_buf.at[slot],
                              s2.at[slot]).wait()
        nxt = nxt_ref[g]

        @pl.when(nxt >= 0)
        def _():
            pltpu.make_async_copy(w1_hbm.at[nxt], w1_buf.at[1 - slot],
                                  s1.at[1 - slot]).start()
            pltpu.make_async_copy(w2_hbm.at[nxt], w2_buf.at[1 - slot],
                                  s2.at[1 - slot]).start()

    @pl.when(be >= 0)
    def _():
        xb = x_ref[...]
        h = jnp.dot(xb, w1_buf[slot], preferred_element_type=jnp.float32)
        h = h + b1_ref[0]
        a = 0.5 * h * (1.0 + jax.lax.erf(h * 0.7071067811865476))
        y = jnp.dot(a, w2_buf[slot], preferred_element_type=jnp.float32)
        y = y + b2_ref[0]
        o_ref[...] = y * wg_ref[...]


def kernel(x, Wr, W1, b1, W2, b2):
    B, S, D = x.shape
    E = Wr.shape[1]
    H = W1.shape[2]
    K = 2
    N = B * S
    P = N * K
    P_MAX = P + E * _T_B
    G_MAX = P_MAX // _T_B

    x2d = x.reshape(N, D)

    probs, sel, w = pl.pallas_call(
        functools.partial(_router_kernel, num_experts=E),
        grid=(N // _T_R,),
        in_specs=[
            pl.BlockSpec((_T_R, D), lambda i: (i, 0)),
            pl.BlockSpec((D, E), lambda i: (0, 0)),
        ],
        out_specs=[
            pl.BlockSpec((_T_R, E), lambda i: (i, 0)),
            pl.BlockSpec((_T_R, K), lambda i: (i, 0)),
            pl.BlockSpec((_T_R, K), lambda i: (i, 0)),
        ],
        out_shape=[
            jax.ShapeDtypeStruct((N, E), jnp.float32),
            jax.ShapeDtypeStruct((N, K), jnp.int32),
            jax.ShapeDtypeStruct((N, K), jnp.float32),
        ],
    )(x2d, Wr)

    e_flat = sel.reshape(P)
    onehot = (e_flat[:, None] == jnp.arange(E, dtype=jnp.int32)[None, :]
              ).astype(jnp.int32)
    ranks_all = jnp.cumsum(onehot, axis=0) - onehot
    rank = jnp.take_along_axis(ranks_all, e_flat[:, None], axis=1)[:, 0]
    counts = jnp.sum(onehot, axis=0)
    padded = ((counts + _T_B - 1) // _T_B) * _T_B
    ends = jnp.cumsum(padded)
    offs = ends - padded
    pos = offs[e_flat] + rank

    blk_start = jnp.arange(G_MAX, dtype=jnp.int32) * _T_B
    be = jnp.sum((blk_start[:, None] >= ends[None, :]).astype(jnp.int32),
                 axis=1)
    block_expert = jnp.where(be < E, be, -1).astype(jnp.int32)

    prev = jnp.concatenate(
        [jnp.full((1,), -2, jnp.int32), block_expert[:-1]])
    isfirst = ((block_expert != prev) & (block_expert >= 0)).astype(jnp.int32)
    slot = ((jnp.cumsum(isfirst) - 1) % 2).astype(jnp.int32)
    gi = jnp.arange(G_MAX, dtype=jnp.int32)
    start_pos = jnp.where(isfirst == 1, gi, G_MAX)
    nxt_start = jnp.flip(jax.lax.cummin(jnp.flip(
        jnp.concatenate([start_pos[1:], jnp.full((1,), G_MAX, jnp.int32)]))))
    nxt_e = jnp.where(nxt_start < G_MAX,
                      block_expert[jnp.minimum(nxt_start, G_MAX - 1)], -1)

    pos2 = pos.reshape(N, K)
    pos0 = pos2[:, 0]
    pos1 = pos2[:, 1]
    w0 = w[:, 0]
    w1 = w[:, 1]

    NW = 32
    TPW = N // NW
    mesh = plsc.VectorSubcoreMesh(core_axis_name="c", subcore_axis_name="s")

    def _dispatch_kernel(x_hbm, p0_hbm, p1_hbm, w0_hbm, w1_hbm,
                         xg_hbm, wg_hbm, rows_v, idx0_v, idx1_v,
                         val0_v, val1_v, sem0, sem1, sem2, sem3):
        wid = lax.axis_index("s") * 2 + lax.axis_index("c")
        base = wid * TPW
        l0 = pltpu.async_copy(p0_hbm.at[pl.ds(base, TPW)], idx0_v, sem0)
        l1 = pltpu.async_copy(p1_hbm.at[pl.ds(base, TPW)], idx1_v, sem1)
        l2 = pltpu.async_copy(w0_hbm.at[pl.ds(base, TPW)], val0_v, sem2)
        l3 = pltpu.async_copy(w1_hbm.at[pl.ds(base, TPW)], val1_v, sem3)
        pltpu.sync_copy(x_hbm.at[pl.ds(base, TPW)], rows_v)
        l0.wait()
        l1.wait()
        l2.wait()
        l3.wait()
        c0 = pltpu.async_copy(rows_v, xg_hbm.at[idx0_v], sem0)
        c1 = pltpu.async_copy(rows_v, xg_hbm.at[idx1_v], sem1)
        c2 = pltpu.async_copy(val0_v, wg_hbm.at[idx0_v], sem2)
        c3 = pltpu.async_copy(val1_v, wg_hbm.at[idx1_v], sem3)
        c0.wait()
        c1.wait()
        c2.wait()
        c3.wait()

    xg, wg = pl.kernel(
        _dispatch_kernel,
        mesh=mesh,
        out_type=[
            jax.ShapeDtypeStruct((P_MAX, D), jnp.float32),
            jax.ShapeDtypeStruct((P_MAX,), jnp.float32),
        ],
        scratch_types=[
            pltpu.VMEM((TPW, D), jnp.float32),
            pltpu.VMEM((TPW,), jnp.int32),
            pltpu.VMEM((TPW,), jnp.int32),
            pltpu.VMEM((TPW,), jnp.float32),
            pltpu.VMEM((TPW,), jnp.float32),
            pltpu.SemaphoreType.DMA,
            pltpu.SemaphoreType.DMA,
            pltpu.SemaphoreType.DMA,
            pltpu.SemaphoreType.DMA,
        ],
    )(x2d, pos0, pos1, w0, w1)

    grid_spec = pltpu.PrefetchScalarGridSpec(
        num_scalar_prefetch=4,
        grid=(G_MAX,),
        in_specs=[
            pl.BlockSpec((_T_B, D), lambda g, be, f, s, n: (g, 0)),
            pl.BlockSpec((1, 1, H),
                         lambda g, be, f, s, n: (jnp.maximum(be[g], 0), 0, 0)),
            pl.BlockSpec((1, 1, D),
                         lambda g, be, f, s, n: (jnp.maximum(be[g], 0), 0, 0)),
            pl.BlockSpec((_T_B, 1), lambda g, be, f, s, n: (g, 0)),
            pl.BlockSpec(memory_space=pl.ANY),
            pl.BlockSpec(memory_space=pl.ANY),
        ],
        out_specs=pl.BlockSpec((_T_B, D), lambda g, be, f, s, n: (g, 0)),
        scratch_shapes=[
            pltpu.VMEM((2, D, H), jnp.float32),
            pltpu.VMEM((2, H, D), jnp.float32),
            pltpu.SemaphoreType.DMA((2,)),
            pltpu.SemaphoreType.DMA((2,)),
        ],
    )
    yg = pl.pallas_call(
        _ffn_kernel,
        grid_spec=grid_spec,
        out_shape=jax.ShapeDtypeStruct((P_MAX, D), jnp.float32),
    )(block_expert, isfirst, slot, nxt_e,
      xg, b1.reshape(E, 1, H), b2.reshape(E, 1, D), wg[:, None], W1, W2)

    SUB = 2
    TPS = TPW // SUB
    CHUNK_V = TPS * D // 16

    def _combine_kernel(yg_hbm, p0_hbm, p1_hbm, o_hbm,
                        i0_v, i1_v, b0_v, b1_v, csem0, csem1):
        wid = lax.axis_index("s") * 2 + lax.axis_index("c")
        for sub in range(SUB):
            base = wid * TPW + sub * TPS
            pltpu.sync_copy(p0_hbm.at[pl.ds(base, TPS)], i0_v)
            pltpu.sync_copy(p1_hbm.at[pl.ds(base, TPS)], i1_v)
            g0 = pltpu.async_copy(yg_hbm.at[i0_v], b0_v, csem0)
            g1 = pltpu.async_copy(yg_hbm.at[i1_v], b1_v, csem1)
            g0.wait()
            g1.wait()

            def _add(i, _):
                r = i // (D // 16)
                c = (i - r * (D // 16)) * 16
                b0_v[r, pl.ds(c, 16)] = (b0_v[r, pl.ds(c, 16)]
                                         + b1_v[r, pl.ds(c, 16)])
                return 0

            lax.fori_loop(0, CHUNK_V, _add, 0, unroll=8)
            pltpu.sync_copy(b0_v, o_hbm.at[pl.ds(base, TPS)])

    out2d = pl.kernel(
        _combine_kernel,
        mesh=mesh,
        out_type=jax.ShapeDtypeStruct((N, D), jnp.float32),
        scratch_types=[
            pltpu.VMEM((TPS,), jnp.int32),
            pltpu.VMEM((TPS,), jnp.int32),
            pltpu.VMEM((TPS, D), jnp.float32),
            pltpu.VMEM((TPS, D), jnp.float32),
            pltpu.SemaphoreType.DMA,
            pltpu.SemaphoreType.DMA,
        ],
    )(yg, pos0, pos1)

    return (out2d.reshape(B, S, D), probs.reshape(B, S, E),
            sel.reshape(B, S, K), w.reshape(B, S, K))
